# baseline (speedup 1.0000x reference)
import functools

import jax
import jax.numpy as jnp
from jax import lax
from jax.experimental import pallas as pl
from jax.experimental.pallas import tpu as pltpu
from jax.experimental.pallas import tpu_sc as plsc

VOCAB = 1000000
SEQ = 200
EMBED = 64
BATCH = 4096

NC = 2
NS = 16
NW = NC * NS
ROWS_PER_W = BATCH * SEQ // NW
LANES = 16
VPR = EMBED // LANES

BLK = 128
NBLK = ROWS_PER_W // BLK
NBUF = 4
AHEAD = 2
POS_REP = 2 * SEQ

_mesh = plsc.VectorSubcoreMesh(core_axis_name="c", subcore_axis_name="s")


@functools.partial(
    pl.kernel,
    out_type=jax.ShapeDtypeStruct((BATCH * SEQ, EMBED), jnp.float32),
    mesh=_mesh,
    compiler_params=pltpu.CompilerParams(use_tc_tiling_on_sc=False),
    scratch_types=[
        pltpu.VMEM((ROWS_PER_W,), jnp.int32),
        pltpu.VMEM((POS_REP, EMBED), jnp.float32),
        [pltpu.VMEM((BLK, EMBED), jnp.float32) for _ in range(NBUF)],
        [pltpu.SemaphoreType.DMA for _ in range(NBUF)],
        [pltpu.SemaphoreType.DMA for _ in range(NBUF)],
    ],
)
def _seq_embed(seq_hbm, tok_hbm, pos_hbm, out_hbm, idx_v, pos_v, bufs,
               gsems, ssems):
    wid = lax.axis_index("s") * NC + lax.axis_index("c")
    base = wid * ROWS_PER_W

    pltpu.sync_copy(seq_hbm.at[pl.ds(base, ROWS_PER_W)], idx_v)
    pltpu.sync_copy(pos_hbm, pos_v.at[pl.ds(0, SEQ)])
    pltpu.sync_copy(pos_hbm, pos_v.at[pl.ds(SEQ, SEQ)])

    def start_gather(b, slot):
        pltpu.async_copy(
            tok_hbm.at[idx_v.at[pl.ds(b * BLK, BLK)]], bufs[slot],
            gsems[slot])

    def wait_gather(slot):
        pltpu.make_async_copy(
            tok_hbm.at[idx_v.at[pl.ds(0, BLK)]], bufs[slot],
            gsems[slot]).wait()

    def start_scatter(b, slot):
        pltpu.async_copy(
            bufs[slot], out_hbm.at[pl.ds(base + b * BLK, BLK)], ssems[slot])

    def wait_scatter(slot):
        pltpu.make_async_copy(
            bufs[slot], out_hbm.at[pl.ds(0, BLK)], ssems[slot]).wait()

    for s in range(AHEAD):
        start_gather(s, s)

    def group(g, carry):
        for s in range(NBUF):
            b = g * NBUF + s
            wait_gather(s)

            p0 = lax.rem(b * BLK, SEQ)
            buf = bufs[s]

            def add_row(j, carry2):
                for k in range(VPR):
                    plsc.addupdate(
                        buf.at[j, pl.ds(k * LANES, LANES)],
                        pos_v[p0 + j, pl.ds(k * LANES, LANES)])
                return carry2

            lax.fori_loop(0, BLK, add_row, 0, unroll=4)

            start_scatter(b, s)

            nb = b + AHEAD
            t = (s + AHEAD) % NBUF

            @pl.when(nb < NBLK)
            def _():
                @pl.when(nb >= NBUF)
                def _():
                    wait_scatter(t)
                start_gather(nb, t)

        return carry

    lax.fori_loop(0, NBLK // NBUF, group, 0)

    for b in range(NBLK - AHEAD, NBLK):
        wait_scatter(b % NBUF)


def kernel(sequence, token_table, pos_table):
    seq_flat = sequence.reshape(-1).astype(jnp.int32)
    out = _seq_embed(seq_flat, token_table, pos_table)
    return out.reshape(BATCH, SEQ, EMBED)
